# baseline (speedup 1.0000x reference)
import jax
import jax.numpy as jnp
from jax import lax
from jax.experimental import pallas as pl
from jax.experimental.pallas import tpu as pltpu
from jax.experimental.pallas import tpu_sc as plsc

N = 100000
C = 128
B = 64
EPS = 1e-05

NTC = 70000
NSC = N - NTC

NW = 32
CH = 200
NCHUNK = NSC // CH
CPW = -(-NCHUNK // NW)
NFULL = NCHUNK - NW * (CPW - 1)
NCG = C // 16

BLK = 5000
TBLK = BLK
NBT = NTC // TBLK
NB = N // BLK


def _sc_stats_body(x_hbm, ids_hbm, out1, out2, outc,
                   xbuf0, xbuf1, idbuf, acc1, acc2, cnt, sem0, sem1, semi):
    w = lax.axis_index("s") * 2 + lax.axis_index("c")

    start = jnp.where(w < NFULL, w * CPW,
                      NFULL * CPW + (w - NFULL) * (CPW - 1))
    count = jnp.where(w < NFULL, CPW, CPW - 1)
    base = jnp.minimum(start, NCHUNK - CPW)
    loff = (start - base) * CH
    idcp = pltpu.make_async_copy(
        ids_hbm.at[pl.ds(NTC + base * CH, CPW * CH)],
        idbuf.at[pl.ds(0, CPW * CH)], semi)
    idcp.start()

    zeros16 = jnp.zeros((16,), jnp.float32)

    bufs = (xbuf0, xbuf1)
    sems = (sem0, sem1)

    def _copy(i, buf, sem):
        return pltpu.make_async_copy(
            x_hbm.at[pl.ds(NTC + (start + i) * CH, CH)], buf, sem)

    _copy(0, xbuf0, sem0).start()

    def _zero(g, _):
        for j in range(NCG):
            acc1[g, pl.ds(16 * j, 16)] = zeros16
            acc2[g, pl.ds(16 * j, 16)] = zeros16
        cnt[g, :] = zeros16
        return 0

    lax.fori_loop(0, B, _zero, 0)
    idcp.wait()

    for i in range(CPW):
        xb = bufs[i % 2]
        sm = sems[i % 2]
        nxb = bufs[(i + 1) % 2]
        nsm = sems[(i + 1) % 2]

        @pl.when(i < count)
        def _step(i=i, xb=xb, sm=sm, nxb=nxb, nsm=nsm):
            _copy(i, xb, sm).wait()

            @pl.when(i + 1 < count)
            def _prefetch():
                _copy(i + 1, nxb, nsm).start()

            ib = loff + i * CH
            first = idbuf[pl.ds(ib, 16)][0]
            last = idbuf[pl.ds(ib + CH - 16, 16)][15]

            @pl.when(first == last)
            def _dense():
                def _row(r, carry):
                    out = []
                    for j in range(NCG):
                        v = xb[r, pl.ds(16 * j, 16)]
                        out.append(carry[2 * j] + v)
                        out.append(carry[2 * j + 1] + v * v)
                    return tuple(out)

                init = (zeros16,) * (2 * NCG)
                sums = lax.fori_loop(0, CH, _row, init, unroll=4)
                for j in range(NCG):
                    plsc.addupdate(acc1.at[first, pl.ds(16 * j, 16)],
                                   sums[2 * j])
                    plsc.addupdate(acc2.at[first, pl.ds(16 * j, 16)],
                                   sums[2 * j + 1])
                plsc.addupdate(cnt.at[first],
                               jnp.full((16,), float(CH), jnp.float32))

            @pl.when(first != last)
            def _rowwise():
                def _row(r, _):
                    b = idbuf[pl.ds(ib + r, 16)][0]
                    for j in range(NCG):
                        v = xb[r, pl.ds(16 * j, 16)]
                        plsc.addupdate(acc1.at[b, pl.ds(16 * j, 16)], v)
                        plsc.addupdate(acc2.at[b, pl.ds(16 * j, 16)], v * v)
                    plsc.addupdate(cnt.at[b],
                                   jnp.full((16,), 1.0, jnp.float32))
                    return 0

                lax.fori_loop(0, CH, _row, 0)

    pltpu.sync_copy(acc1, out1.at[w])
    pltpu.sync_copy(acc2, out2.at[w])
    pltpu.sync_copy(cnt, outc.at[w])


def _sc_stats(x, ids):
    mesh = plsc.VectorSubcoreMesh(core_axis_name="c", subcore_axis_name="s")
    f32 = jnp.float32
    run = pl.kernel(
        _sc_stats_body,
        out_type=[
            jax.ShapeDtypeStruct((NW, B, C), f32),
            jax.ShapeDtypeStruct((NW, B, C), f32),
            jax.ShapeDtypeStruct((NW, B, 16), f32),
        ],
        mesh=mesh,
        scratch_types=[
            pltpu.VMEM((CH, C), f32),
            pltpu.VMEM((CH, C), f32),
            pltpu.VMEM((CPW * CH + 16,), jnp.int32),
            pltpu.VMEM((B, C), f32),
            pltpu.VMEM((B, C), f32),
            pltpu.VMEM((B, 16), f32),
            pltpu.SemaphoreType.DMA,
            pltpu.SemaphoreType.DMA,
            pltpu.SemaphoreType.DMA,
        ],
    )
    return run(x, ids)


def _tc_stats_kernel(x_ref, ids_ref, s1_out, s2_out, cnt_out,
                     s1_ref, s2_ref, cnt_ref):
    i = pl.program_id(0)

    @pl.when(i == 0)
    def _init():
        s1_ref[...] = jnp.zeros_like(s1_ref)
        s2_ref[...] = jnp.zeros_like(s2_ref)
        cnt_ref[...] = jnp.zeros_like(cnt_ref)

    x = x_ref[...]
    ids = ids_ref[0]
    onehot_t = (lax.broadcasted_iota(jnp.int32, (B, TBLK), 0) == ids
                ).astype(jnp.float32)
    s1_ref[...] += jnp.dot(onehot_t, x, preferred_element_type=jnp.float32)
    s2_ref[...] += jnp.dot(onehot_t, x * x,
                           preferred_element_type=jnp.float32)
    cnt_ref[...] += jnp.sum(onehot_t, axis=1, keepdims=True)

    @pl.when(i == NBT - 1)
    def _finish():
        s1_out[...] = s1_ref[...]
        s2_out[...] = s2_ref[...]
        cnt_out[...] = cnt_ref[...]


def _norm_kernel(x_ref, ids_ref, p1_ref, p2_ref, pc_ref,
                 t1_ref, t2_ref, tc_ref,
                 w_ref, b_ref, ms_ref, o_ref, a_s, b_s):
    i = pl.program_id(0)

    @pl.when(i == 0)
    def _tables():
        s1 = jnp.sum(p1_ref[...], axis=0) + t1_ref[...]
        s2 = jnp.sum(p2_ref[...], axis=0) + t2_ref[...]
        cntv = jnp.sum(pc_ref[...], axis=0)
        cnt = jnp.maximum(cntv[:, 0:1] + tc_ref[...], 1.0)
        mean = s1 / cnt
        m2 = mean * ms_ref[...]
        var = s2 / cnt - 2.0 * m2 * mean + m2 * m2
        rstd = lax.rsqrt(var + EPS)
        a = w_ref[...] * rstd
        a_s[...] = a
        b_s[...] = b_ref[...] - a * m2

    ids = ids_ref[0]
    onehot_t = (lax.broadcasted_iota(jnp.int32, (B, BLK), 0) == ids
                ).astype(jnp.float32)
    dn = (((0,), (0,)), ((), ()))
    ga = lax.dot_general(onehot_t, a_s[...], dn,
                         preferred_element_type=jnp.float32)
    gb = lax.dot_general(onehot_t, b_s[...], dn,
                         preferred_element_type=jnp.float32)
    o_ref[...] = ga * x_ref[...] + gb


@jax.jit
def kernel(x, batch, weight, bias, mean_scale):
    ids = batch.astype(jnp.int32)
    p1, p2, pc = _sc_stats(x, ids)

    ids3 = ids.reshape(N // TBLK, 1, TBLK)
    t1, t2, tc = pl.pallas_call(
        _tc_stats_kernel,
        grid=(NBT,),
        in_specs=[
            pl.BlockSpec((TBLK, C), lambda i: (i, 0)),
            pl.BlockSpec((1, 1, TBLK), lambda i: (i, 0, 0)),
        ],
        out_specs=[
            pl.BlockSpec((B, C), lambda i: (0, 0)),
            pl.BlockSpec((B, C), lambda i: (0, 0)),
            pl.BlockSpec((B, 1), lambda i: (0, 0)),
        ],
        out_shape=[
            jax.ShapeDtypeStruct((B, C), jnp.float32),
            jax.ShapeDtypeStruct((B, C), jnp.float32),
            jax.ShapeDtypeStruct((B, 1), jnp.float32),
        ],
        scratch_shapes=[
            pltpu.VMEM((B, C), jnp.float32),
            pltpu.VMEM((B, C), jnp.float32),
            pltpu.VMEM((B, 1), jnp.float32),
        ],
    )(x, ids3)

    w2 = weight.reshape(1, C)
    b2 = bias.reshape(1, C)
    ms2 = mean_scale.reshape(1, C)

    out = pl.pallas_call(
        _norm_kernel,
        grid=(NB,),
        in_specs=[
            pl.BlockSpec((BLK, C), lambda i: (i, 0)),
            pl.BlockSpec((1, 1, BLK), lambda i: (i, 0, 0)),
            pl.BlockSpec((NW, B, C), lambda i: (0, 0, 0)),
            pl.BlockSpec((NW, B, C), lambda i: (0, 0, 0)),
            pl.BlockSpec((NW, B, 16), lambda i: (0, 0, 0)),
            pl.BlockSpec((B, C), lambda i: (0, 0)),
            pl.BlockSpec((B, C), lambda i: (0, 0)),
            pl.BlockSpec((B, 1), lambda i: (0, 0)),
            pl.BlockSpec((1, C), lambda i: (0, 0)),
            pl.BlockSpec((1, C), lambda i: (0, 0)),
            pl.BlockSpec((1, C), lambda i: (0, 0)),
        ],
        out_specs=pl.BlockSpec((BLK, C), lambda i: (i, 0)),
        out_shape=jax.ShapeDtypeStruct((N, C), jnp.float32),
        scratch_shapes=[
            pltpu.VMEM((B, C), jnp.float32),
            pltpu.VMEM((B, C), jnp.float32),
        ],
    )(x, ids3, p1, p2, pc, t1, t2, tc, w2, b2, ms2)
    return out


# speedup vs baseline: 1.1846x; 1.1846x over previous
import jax
import jax.numpy as jnp
from jax import lax
from jax.experimental import pallas as pl
from jax.experimental.pallas import tpu as pltpu
from jax.experimental.pallas import tpu_sc as plsc

N = 100000
C = 128
B = 64
EPS = 1e-05

NTC = 80000
NSC = N - NTC

NW = 32
CH = 200
NCHUNK = NSC // CH
CPW = -(-NCHUNK // NW)
NFULL = NCHUNK - NW * (CPW - 1)
NCG = C // 16

BLK = 10000
TBLK = BLK
NBT = NTC // TBLK
NB = N // BLK


def _sc_stats_body(x_hbm, ids_hbm, out1, out2, outc,
                   xbuf0, xbuf1, idbuf, acc1, acc2, cnt, sem0, sem1, semi):
    w = lax.axis_index("s") * 2 + lax.axis_index("c")

    start = jnp.where(w < NFULL, w * CPW,
                      NFULL * CPW + (w - NFULL) * (CPW - 1))
    count = jnp.where(w < NFULL, CPW, CPW - 1)
    base = jnp.minimum(start, NCHUNK - CPW)
    loff = (start - base) * CH
    idcp = pltpu.make_async_copy(
        ids_hbm.at[pl.ds(NTC + base * CH, CPW * CH)],
        idbuf.at[pl.ds(0, CPW * CH)], semi)
    idcp.start()

    zeros16 = jnp.zeros((16,), jnp.float32)

    bufs = (xbuf0, xbuf1)
    sems = (sem0, sem1)

    def _copy(i, buf, sem):
        return pltpu.make_async_copy(
            x_hbm.at[pl.ds(NTC + (start + i) * CH, CH)], buf, sem)

    _copy(0, xbuf0, sem0).start()

    def _zero(g, _):
        for j in range(NCG):
            acc1[g, pl.ds(16 * j, 16)] = zeros16
            acc2[g, pl.ds(16 * j, 16)] = zeros16
        cnt[g, :] = zeros16
        return 0

    lax.fori_loop(0, B, _zero, 0)
    idcp.wait()

    for i in range(CPW):
        xb = bufs[i % 2]
        sm = sems[i % 2]
        nxb = bufs[(i + 1) % 2]
        nsm = sems[(i + 1) % 2]

        @pl.when(i < count)
        def _step(i=i, xb=xb, sm=sm, nxb=nxb, nsm=nsm):
            _copy(i, xb, sm).wait()

            @pl.when(i + 1 < count)
            def _prefetch():
                _copy(i + 1, nxb, nsm).start()

            ib = loff + i * CH
            first = idbuf[pl.ds(ib, 16)][0]
            last = idbuf[pl.ds(ib + CH - 16, 16)][15]

            @pl.when(first == last)
            def _dense():
                def _row(r, carry):
                    out = []
                    for j in range(NCG):
                        v = xb[r, pl.ds(16 * j, 16)]
                        out.append(carry[2 * j] + v)
                        out.append(carry[2 * j + 1] + v * v)
                    return tuple(out)

                init = (zeros16,) * (2 * NCG)
                sums = lax.fori_loop(0, CH, _row, init, unroll=4)
                for j in range(NCG):
                    plsc.addupdate(acc1.at[first, pl.ds(16 * j, 16)],
                                   sums[2 * j])
                    plsc.addupdate(acc2.at[first, pl.ds(16 * j, 16)],
                                   sums[2 * j + 1])
                plsc.addupdate(cnt.at[first],
                               jnp.full((16,), float(CH), jnp.float32))

            @pl.when(first != last)
            def _rowwise():
                def _row(r, _):
                    b = idbuf[pl.ds(ib + r, 16)][0]
                    for j in range(NCG):
                        v = xb[r, pl.ds(16 * j, 16)]
                        plsc.addupdate(acc1.at[b, pl.ds(16 * j, 16)], v)
                        plsc.addupdate(acc2.at[b, pl.ds(16 * j, 16)], v * v)
                    plsc.addupdate(cnt.at[b],
                                   jnp.full((16,), 1.0, jnp.float32))
                    return 0

                lax.fori_loop(0, CH, _row, 0)

    pltpu.sync_copy(acc1, out1.at[w])
    pltpu.sync_copy(acc2, out2.at[w])
    pltpu.sync_copy(cnt, outc.at[w])


def _sc_stats(x, ids):
    mesh = plsc.VectorSubcoreMesh(core_axis_name="c", subcore_axis_name="s")
    f32 = jnp.float32
    run = pl.kernel(
        _sc_stats_body,
        out_type=[
            jax.ShapeDtypeStruct((NW, B, C), f32),
            jax.ShapeDtypeStruct((NW, B, C), f32),
            jax.ShapeDtypeStruct((NW, B, 16), f32),
        ],
        mesh=mesh,
        scratch_types=[
            pltpu.VMEM((CH, C), f32),
            pltpu.VMEM((CH, C), f32),
            pltpu.VMEM((CPW * CH + 16,), jnp.int32),
            pltpu.VMEM((B, C), f32),
            pltpu.VMEM((B, C), f32),
            pltpu.VMEM((B, 16), f32),
            pltpu.SemaphoreType.DMA,
            pltpu.SemaphoreType.DMA,
            pltpu.SemaphoreType.DMA,
        ],
    )
    return run(x, ids)


def _tc_stats_kernel(x_ref, ids_ref, s1_out, s2_out, cnt_out,
                     s1_ref, s2_ref, cnt_ref):
    i = pl.program_id(0)

    @pl.when(i == 0)
    def _init():
        s1_ref[...] = jnp.zeros_like(s1_ref)
        s2_ref[...] = jnp.zeros_like(s2_ref)
        cnt_ref[...] = jnp.zeros_like(cnt_ref)

    x = x_ref[...]
    ids = ids_ref[0]
    onehot_t = (lax.broadcasted_iota(jnp.int32, (B, TBLK), 0) == ids
                ).astype(jnp.float32)
    s1_ref[...] += jnp.dot(onehot_t, x, preferred_element_type=jnp.float32)
    s2_ref[...] += jnp.dot(onehot_t, x * x,
                           preferred_element_type=jnp.float32)
    cnt_ref[...] += jnp.sum(onehot_t, axis=1, keepdims=True)

    @pl.when(i == NBT - 1)
    def _finish():
        s1_out[...] = s1_ref[...]
        s2_out[...] = s2_ref[...]
        cnt_out[...] = cnt_ref[...]


def _norm_kernel(x_ref, ids_ref, p1_ref, p2_ref, pc_ref,
                 t1_ref, t2_ref, tc_ref,
                 w_ref, b_ref, ms_ref, o_ref, a_s, b_s):
    i = pl.program_id(0)

    @pl.when(i == 0)
    def _tables():
        s1 = jnp.sum(p1_ref[...], axis=0) + t1_ref[...]
        s2 = jnp.sum(p2_ref[...], axis=0) + t2_ref[...]
        cntv = jnp.sum(pc_ref[...], axis=0)
        cnt = jnp.maximum(cntv[:, 0:1] + tc_ref[...], 1.0)
        mean = s1 / cnt
        m2 = mean * ms_ref[...]
        var = s2 / cnt - 2.0 * m2 * mean + m2 * m2
        rstd = lax.rsqrt(var + EPS)
        a = w_ref[...] * rstd
        a_s[...] = a
        b_s[...] = b_ref[...] - a * m2

    ids = ids_ref[0]
    onehot_t = (lax.broadcasted_iota(jnp.int32, (B, BLK), 0) == ids
                ).astype(jnp.float32)
    dn = (((0,), (0,)), ((), ()))
    ga = lax.dot_general(onehot_t, a_s[...], dn,
                         preferred_element_type=jnp.float32)
    gb = lax.dot_general(onehot_t, b_s[...], dn,
                         preferred_element_type=jnp.float32)
    o_ref[...] = ga * x_ref[...] + gb


@jax.jit
def kernel(x, batch, weight, bias, mean_scale):
    ids = batch.astype(jnp.int32)
    p1, p2, pc = _sc_stats(x, ids)

    ids3 = ids.reshape(N // TBLK, 1, TBLK)
    t1, t2, tc = pl.pallas_call(
        _tc_stats_kernel,
        grid=(NBT,),
        in_specs=[
            pl.BlockSpec((TBLK, C), lambda i: (i, 0)),
            pl.BlockSpec((1, 1, TBLK), lambda i: (i, 0, 0)),
        ],
        out_specs=[
            pl.BlockSpec((B, C), lambda i: (0, 0)),
            pl.BlockSpec((B, C), lambda i: (0, 0)),
            pl.BlockSpec((B, 1), lambda i: (0, 0)),
        ],
        out_shape=[
            jax.ShapeDtypeStruct((B, C), jnp.float32),
            jax.ShapeDtypeStruct((B, C), jnp.float32),
            jax.ShapeDtypeStruct((B, 1), jnp.float32),
        ],
        scratch_shapes=[
            pltpu.VMEM((B, C), jnp.float32),
            pltpu.VMEM((B, C), jnp.float32),
            pltpu.VMEM((B, 1), jnp.float32),
        ],
    )(x, ids3)

    w2 = weight.reshape(1, C)
    b2 = bias.reshape(1, C)
    ms2 = mean_scale.reshape(1, C)

    out = pl.pallas_call(
        _norm_kernel,
        grid=(NB,),
        in_specs=[
            pl.BlockSpec((BLK, C), lambda i: (i, 0)),
            pl.BlockSpec((1, 1, BLK), lambda i: (i, 0, 0)),
            pl.BlockSpec((NW, B, C), lambda i: (0, 0, 0)),
            pl.BlockSpec((NW, B, C), lambda i: (0, 0, 0)),
            pl.BlockSpec((NW, B, 16), lambda i: (0, 0, 0)),
            pl.BlockSpec((B, C), lambda i: (0, 0)),
            pl.BlockSpec((B, C), lambda i: (0, 0)),
            pl.BlockSpec((B, 1), lambda i: (0, 0)),
            pl.BlockSpec((1, C), lambda i: (0, 0)),
            pl.BlockSpec((1, C), lambda i: (0, 0)),
            pl.BlockSpec((1, C), lambda i: (0, 0)),
        ],
        out_specs=pl.BlockSpec((BLK, C), lambda i: (i, 0)),
        out_shape=jax.ShapeDtypeStruct((N, C), jnp.float32),
        scratch_shapes=[
            pltpu.VMEM((B, C), jnp.float32),
            pltpu.VMEM((B, C), jnp.float32),
        ],
    )(x, ids3, p1, p2, pc, t1, t2, tc, w2, b2, ms2)
    return out
